# baseline (speedup 1.0000x reference)
import functools

import jax
import jax.numpy as jnp
from jax.experimental import pallas as pl

F32 = jnp.float32
_HI = jax.lax.Precision.HIGHEST


def _mm_stats_kernel(x_ref, w_ref, b_ref, y_ref, s_ref):
    y = jnp.dot(x_ref[...].astype(jnp.bfloat16),
                w_ref[...].astype(jnp.bfloat16),
                preferred_element_type=F32) + b_ref[0:1, :]
    y_ref[...] = y

    @pl.when(pl.program_id(0) == 0)
    def _init():
        s_ref[...] = jnp.zeros_like(s_ref)

    s_ref[0:1, :] += jnp.sum(y, axis=0, keepdims=True)
    s_ref[1:2, :] += jnp.sum(y * y, axis=0, keepdims=True)


def _linear_with_stats(x2d, W, b, blk):
    T, Cin = x2d.shape
    Cout = W.shape[1]
    y, stats = pl.pallas_call(
        _mm_stats_kernel,
        grid=(T // blk,),
        in_specs=[
            pl.BlockSpec((blk, Cin), lambda i: (i, 0)),
            pl.BlockSpec((Cin, Cout), lambda i: (0, 0)),
            pl.BlockSpec((1, Cout), lambda i: (0, 0)),
        ],
        out_specs=[
            pl.BlockSpec((blk, Cout), lambda i: (i, 0)),
            pl.BlockSpec((8, Cout), lambda i: (0, 0)),
        ],
        out_shape=[
            jax.ShapeDtypeStruct((T, Cout), F32),
            jax.ShapeDtypeStruct((8, Cout), F32),
        ],
    )(x2d, W, b.reshape(1, Cout))
    return y, stats


def _chop_bf16(x):
    u = jax.lax.bitcast_convert_type(x, jnp.uint32) & jnp.uint32(0xFFFF0000)
    return jax.lax.bitcast_convert_type(u, F32).astype(jnp.bfloat16)


def _interp_kernel(pu_ref, pdt_ref, ramp_ref, yd_ref, yu_ref, aff_ref, o_ref):
    pdt = pdt_ref[0]
    pu = pu_ref[0]
    xd = jnp.maximum(yd_ref[0] * aff_ref[0:1, :] + aff_ref[1:2, :], 0.0)
    n = pdt.shape[1]
    xd_aug = jnp.concatenate(
        [xd.astype(jnp.bfloat16), jnp.ones((n, 128), jnp.bfloat16)],
        axis=1)
    pd2 = (pdt[0:1] * pdt[0:1] + pdt[1:2] * pdt[1:2]) + pdt[2:3] * pdt[2:3]
    pu2 = ((pu[:, 0:1] * pu[:, 0:1] + pu[:, 1:2] * pu[:, 1:2])
           + pu[:, 2:3] * pu[:, 2:3])
    d2 = pu2 + pd2 - 2.0 * jnp.dot(pu.astype(jnp.bfloat16),
                                   pdt.astype(jnp.bfloat16),
                                   preferred_element_type=F32)
    sel = jnp.where(d2 <= 0.0, ramp_ref[0:1, :], d2)
    t = jnp.min(sel, axis=1, keepdims=True)
    for _ in range(3):
        t = jnp.min(jnp.where(sel > t, sel, jnp.inf), axis=1, keepdims=True)
    w = jnp.where(sel <= t, 1.0 / (jnp.maximum(d2, 0.0) + 1e-16), 0.0)
    acc = jnp.dot(w.astype(jnp.bfloat16), xd_aug,
                  preferred_element_type=F32)
    C = o_ref.shape[2]
    interp = acc[:, :C] / acc[:, C:C + 1]
    xu = jnp.maximum(yu_ref[0] * aff_ref[2:3, :] + aff_ref[3:4, :], 0.0)
    o_ref[0] = interp + xu


def kernel(x_down, x_up, p_down, p_up, W_down, b_down, gamma_down, beta_down,
           W_up, b_up, gamma_up, beta_up):
    B, n, Cin = x_down.shape
    N = x_up.shape[1]
    C = W_down.shape[1]

    y_down, stats_d = _linear_with_stats(
        x_down.reshape(B * n, Cin), W_down, b_down, blk=1024)
    y_up, stats_u = _linear_with_stats(
        x_up.reshape(B * N, C), W_up, b_up, blk=2048)

    def affine(stats, gamma, beta, cnt):
        mean = stats[0] / cnt
        var = stats[1] / cnt - mean * mean
        scale = gamma / jnp.sqrt(var + 1e-5)
        return scale, beta - mean * scale

    sd, td = affine(stats_d, gamma_down, beta_down, float(B * n))
    su, tu = affine(stats_u, gamma_up, beta_up, float(B * N))
    aff = jnp.concatenate(
        [sd[None], td[None], su[None], tu[None],
         jnp.zeros((4, C), F32)], axis=0)

    pu_pad = jnp.pad(p_up, ((0, 0), (0, 0), (0, 5)))
    pd_t = jnp.pad(jnp.swapaxes(p_down, 1, 2), ((0, 0), (0, 5), (0, 0)))
    ramp = jnp.broadcast_to(
        jnp.arange(n, dtype=F32)[None, :] * jnp.float32(1.2e-38), (8, n))

    blk = 2048
    out = pl.pallas_call(
        _interp_kernel,
        grid=(B, N // blk),
        in_specs=[
            pl.BlockSpec((1, blk, 8), lambda b, i: (b, i, 0)),
            pl.BlockSpec((1, 8, n), lambda b, i: (b, 0, 0)),
            pl.BlockSpec((8, n), lambda b, i: (0, 0)),
            pl.BlockSpec((1, n, C), lambda b, i: (b, 0, 0)),
            pl.BlockSpec((1, blk, C), lambda b, i: (b, i, 0)),
            pl.BlockSpec((8, C), lambda b, i: (0, 0)),
        ],
        out_specs=pl.BlockSpec((1, blk, C), lambda b, i: (b, i, 0)),
        out_shape=jax.ShapeDtypeStruct((B, N, C), F32),
    )(pu_pad, pd_t, ramp, y_down.reshape(B, n, C), y_up.reshape(B, N, C), aff)
    return out


# speedup vs baseline: 1.0979x; 1.0979x over previous
import functools

import jax
import jax.numpy as jnp
from jax.experimental import pallas as pl
from jax.experimental.pallas import tpu as pltpu

F32 = jnp.float32
_HI = jax.lax.Precision.HIGHEST


def _mm_stats_kernel(x_ref, w_ref, b_ref, y_ref, s_ref):
    y = jnp.dot(x_ref[...].astype(jnp.bfloat16),
                w_ref[...].astype(jnp.bfloat16),
                preferred_element_type=F32) + b_ref[0:1, :]
    y_ref[...] = y

    @pl.when(pl.program_id(0) == 0)
    def _init():
        s_ref[...] = jnp.zeros_like(s_ref)

    s_ref[0:1, :] += jnp.sum(y, axis=0, keepdims=True)
    s_ref[1:2, :] += jnp.sum(y * y, axis=0, keepdims=True)


def _linear_with_stats(x2d, W, b, blk):
    T, Cin = x2d.shape
    Cout = W.shape[1]
    y, stats = pl.pallas_call(
        _mm_stats_kernel,
        grid=(T // blk,),
        in_specs=[
            pl.BlockSpec((blk, Cin), lambda i: (i, 0)),
            pl.BlockSpec((Cin, Cout), lambda i: (0, 0)),
            pl.BlockSpec((1, Cout), lambda i: (0, 0)),
        ],
        out_specs=[
            pl.BlockSpec((blk, Cout), lambda i: (i, 0)),
            pl.BlockSpec((8, Cout), lambda i: (0, 0)),
        ],
        out_shape=[
            jax.ShapeDtypeStruct((T, Cout), F32),
            jax.ShapeDtypeStruct((8, Cout), F32),
        ],
    )(x2d, W, b.reshape(1, Cout))
    return y, stats


def _up_moments_kernel(x_ref, w_ref, o_ref, s_acc, xtx_acc):
    xb = x_ref[...].astype(jnp.bfloat16)

    @pl.when(pl.program_id(0) == 0)
    def _init():
        s_acc[...] = jnp.zeros_like(s_acc)
        xtx_acc[...] = jnp.zeros_like(xtx_acc)

    s_acc[0:1, :] += jnp.sum(x_ref[...], axis=0, keepdims=True)
    xtx_acc[...] += jnp.dot(xb.T, xb, preferred_element_type=F32)

    @pl.when(pl.program_id(0) == pl.num_programs(0) - 1)
    def _fin():
        W = w_ref[...]
        q = jnp.sum(W * jnp.dot(xtx_acc[...], W, preferred_element_type=F32,
                                precision=_HI), axis=0, keepdims=True)
        m = jnp.dot(s_acc[0:1, :], W, preferred_element_type=F32,
                    precision=_HI)
        o_ref[0:1, :] = s_acc[0:1, :]
        o_ref[1:2, :] = q
        o_ref[2:3, :] = m


def _up_moments(x2d, W, blk):
    T, C = x2d.shape
    return pl.pallas_call(
        _up_moments_kernel,
        grid=(T // blk,),
        in_specs=[
            pl.BlockSpec((blk, C), lambda i: (i, 0)),
            pl.BlockSpec((C, C), lambda i: (0, 0)),
        ],
        out_specs=pl.BlockSpec((8, C), lambda i: (0, 0)),
        out_shape=jax.ShapeDtypeStruct((8, C), F32),
        scratch_shapes=[pltpu.VMEM((8, C), F32), pltpu.VMEM((C, C), F32)],
    )(x2d, W)


def _chop_bf16(x):
    u = jax.lax.bitcast_convert_type(x, jnp.uint32) & jnp.uint32(0xFFFF0000)
    return jax.lax.bitcast_convert_type(u, F32).astype(jnp.bfloat16)


def _interp_kernel(pu_ref, pdt_ref, ramp_ref, yd_ref, xu_ref, wu_ref,
                   aff_ref, o_ref):
    pdt = pdt_ref[0]
    pu = pu_ref[0]
    xd = jnp.maximum(yd_ref[0] * aff_ref[0:1, :] + aff_ref[1:2, :], 0.0)
    n = pdt.shape[1]
    xd_aug = jnp.concatenate(
        [xd.astype(jnp.bfloat16), jnp.ones((n, 128), jnp.bfloat16)],
        axis=1)
    pd2 = (pdt[0:1] * pdt[0:1] + pdt[1:2] * pdt[1:2]) + pdt[2:3] * pdt[2:3]
    pu2 = ((pu[:, 0:1] * pu[:, 0:1] + pu[:, 1:2] * pu[:, 1:2])
           + pu[:, 2:3] * pu[:, 2:3])
    d2 = pu2 + pd2 - 2.0 * jnp.dot(pu.astype(jnp.bfloat16),
                                   pdt.astype(jnp.bfloat16),
                                   preferred_element_type=F32)
    sel = jnp.where(d2 <= 0.0, ramp_ref[0:1, :], d2)
    t = jnp.min(sel, axis=1, keepdims=True)
    for _ in range(3):
        t = jnp.min(jnp.where(sel > t, sel, jnp.inf), axis=1, keepdims=True)
    w = jnp.where(sel <= t, 1.0 / (jnp.maximum(d2, 0.0) + 1e-16), 0.0)
    acc = jnp.dot(w.astype(jnp.bfloat16), xd_aug,
                  preferred_element_type=F32)
    C = o_ref.shape[2]
    interp = acc[:, :C] / acc[:, C:C + 1]
    yu = jnp.dot(xu_ref[0].astype(jnp.bfloat16),
                 wu_ref[...].astype(jnp.bfloat16), preferred_element_type=F32)
    xu = jnp.maximum(yu * aff_ref[2:3, :] + aff_ref[3:4, :], 0.0)
    o_ref[0] = interp + xu


def kernel(x_down, x_up, p_down, p_up, W_down, b_down, gamma_down, beta_down,
           W_up, b_up, gamma_up, beta_up):
    B, n, Cin = x_down.shape
    N = x_up.shape[1]
    C = W_down.shape[1]

    y_down, stats_d = _linear_with_stats(
        x_down.reshape(B * n, Cin), W_down, b_down, blk=1024)
    mom_u = _up_moments(x_up.reshape(B * N, C), W_up, blk=2048)

    def affine(stats, gamma, beta, cnt):
        mean = stats[0] / cnt
        var = stats[1] / cnt - mean * mean
        scale = gamma / jnp.sqrt(var + 1e-5)
        return scale, beta - mean * scale

    sd, td = affine(stats_d, gamma_down, beta_down, float(B * n))
    m_u = mom_u[2] / float(B * N)
    var_u = mom_u[1] / float(B * N) - m_u * m_u
    su = gamma_up / jnp.sqrt(var_u + 1e-5)
    tu = beta_up - m_u * su
    aff = jnp.concatenate(
        [sd[None], td[None], su[None], tu[None],
         jnp.zeros((4, C), F32)], axis=0)

    pu_pad = jnp.pad(p_up, ((0, 0), (0, 0), (0, 5)))
    pd_t = jnp.pad(jnp.swapaxes(p_down, 1, 2), ((0, 0), (0, 5), (0, 0)))
    ramp = jnp.broadcast_to(
        jnp.arange(n, dtype=F32)[None, :] * jnp.float32(1.2e-38), (8, n))

    blk = 1024
    out = pl.pallas_call(
        _interp_kernel,
        grid=(B, N // blk),
        in_specs=[
            pl.BlockSpec((1, blk, 8), lambda b, i: (b, i, 0)),
            pl.BlockSpec((1, 8, n), lambda b, i: (b, 0, 0)),
            pl.BlockSpec((8, n), lambda b, i: (0, 0)),
            pl.BlockSpec((1, n, C), lambda b, i: (b, 0, 0)),
            pl.BlockSpec((1, blk, C), lambda b, i: (b, i, 0)),
            pl.BlockSpec((C, C), lambda b, i: (0, 0)),
            pl.BlockSpec((8, C), lambda b, i: (0, 0)),
        ],
        out_specs=pl.BlockSpec((1, blk, C), lambda b, i: (b, i, 0)),
        out_shape=jax.ShapeDtypeStruct((B, N, C), F32),
    )(pu_pad, pd_t, ramp, y_down.reshape(B, n, C), x_up, W_up, aff)
    return out


# speedup vs baseline: 1.0980x; 1.0001x over previous
import jax
import jax.numpy as jnp
from jax.experimental import pallas as pl
from jax.experimental.pallas import tpu as pltpu

F32 = jnp.float32
_HI = jax.lax.Precision.HIGHEST


def _mm_stats_kernel(x_ref, w_ref, b_ref, y_ref, s_ref):
    y = jnp.dot(x_ref[...].astype(jnp.bfloat16),
                w_ref[...].astype(jnp.bfloat16),
                preferred_element_type=F32) + b_ref[0:1, :]
    y_ref[...] = y

    @pl.when(pl.program_id(0) == 0)
    def _init():
        s_ref[...] = jnp.zeros_like(s_ref)

    s_ref[0:1, :] += jnp.sum(y, axis=0, keepdims=True)
    s_ref[1:2, :] += jnp.sum(y * y, axis=0, keepdims=True)


def _linear_with_stats(x2d, W, b, blk):
    T, Cin = x2d.shape
    Cout = W.shape[1]
    y, stats = pl.pallas_call(
        _mm_stats_kernel,
        grid=(T // blk,),
        in_specs=[
            pl.BlockSpec((blk, Cin), lambda i: (i, 0)),
            pl.BlockSpec((Cin, Cout), lambda i: (0, 0)),
            pl.BlockSpec((1, Cout), lambda i: (0, 0)),
        ],
        out_specs=[
            pl.BlockSpec((blk, Cout), lambda i: (i, 0)),
            pl.BlockSpec((8, Cout), lambda i: (0, 0)),
        ],
        out_shape=[
            jax.ShapeDtypeStruct((T, Cout), F32),
            jax.ShapeDtypeStruct((8, Cout), F32),
        ],
    )(x2d, W, b.reshape(1, Cout))
    return y, stats


def _up_moments_kernel(x_ref, w_ref, o_ref, s_acc, xtx_acc):
    xb = x_ref[...].astype(jnp.bfloat16)

    @pl.when(pl.program_id(0) == 0)
    def _init():
        s_acc[...] = jnp.zeros_like(s_acc)
        xtx_acc[...] = jnp.zeros_like(xtx_acc)

    s_acc[0:1, :] += jnp.sum(x_ref[...], axis=0, keepdims=True)
    xtx_acc[...] += jnp.dot(xb.T, xb, preferred_element_type=F32)

    @pl.when(pl.program_id(0) == pl.num_programs(0) - 1)
    def _fin():
        W = w_ref[...]
        q = jnp.sum(W * jnp.dot(xtx_acc[...], W, preferred_element_type=F32,
                                precision=_HI), axis=0, keepdims=True)
        m = jnp.dot(s_acc[0:1, :], W, preferred_element_type=F32,
                    precision=_HI)
        o_ref[0:1, :] = s_acc[0:1, :]
        o_ref[1:2, :] = q
        o_ref[2:3, :] = m


def _up_moments(x2d, W, blk):
    T, C = x2d.shape
    return pl.pallas_call(
        _up_moments_kernel,
        grid=(T // blk,),
        in_specs=[
            pl.BlockSpec((blk, C), lambda i: (i, 0)),
            pl.BlockSpec((C, C), lambda i: (0, 0)),
        ],
        out_specs=pl.BlockSpec((8, C), lambda i: (0, 0)),
        out_shape=jax.ShapeDtypeStruct((8, C), F32),
        scratch_shapes=[pltpu.VMEM((8, C), F32), pltpu.VMEM((C, C), F32)],
    )(x2d, W)


def _interp_kernel(pu_ref, pdt_ref, ramp_ref, yd_ref, xu_ref, wu_ref,
                   aff_ref, o_ref):
    pdt = pdt_ref[0]
    pu = pu_ref[0]
    xd = jnp.maximum(yd_ref[0] * aff_ref[0:1, :] + aff_ref[1:2, :], 0.0)
    n = pdt.shape[1]
    xd_aug = jnp.concatenate(
        [xd.astype(jnp.bfloat16), jnp.ones((n, 128), jnp.bfloat16)],
        axis=1)
    pd2 = (pdt[0:1] * pdt[0:1] + pdt[1:2] * pdt[1:2]) + pdt[2:3] * pdt[2:3]
    pu2 = ((pu[:, 0:1] * pu[:, 0:1] + pu[:, 1:2] * pu[:, 1:2])
           + pu[:, 2:3] * pu[:, 2:3])
    d2 = pu2 + pd2 - 2.0 * jnp.dot(pu.astype(jnp.bfloat16),
                                   pdt.astype(jnp.bfloat16),
                                   preferred_element_type=F32)
    sel = jnp.where(d2 <= 0.0, ramp_ref[0:1, :], d2)
    t = jnp.min(sel, axis=1, keepdims=True)
    for _ in range(3):
        t = jnp.min(jnp.where(sel > t, sel, jnp.inf), axis=1, keepdims=True)
    w = jnp.where(sel <= t, 1.0 / (jnp.maximum(d2, 0.0) + 1e-16), 0.0)
    acc = jnp.dot(w.astype(jnp.bfloat16), xd_aug,
                  preferred_element_type=F32)
    C = o_ref.shape[2]
    interp = acc[:, :C] / acc[:, C:C + 1]
    yu = jnp.dot(xu_ref[0].astype(jnp.bfloat16),
                 wu_ref[...].astype(jnp.bfloat16), preferred_element_type=F32)
    xu = jnp.maximum(yu * aff_ref[2:3, :] + aff_ref[3:4, :], 0.0)
    o_ref[0] = interp + xu


def kernel(x_down, x_up, p_down, p_up, W_down, b_down, gamma_down, beta_down,
           W_up, b_up, gamma_up, beta_up):
    B, n, Cin = x_down.shape
    N = x_up.shape[1]
    C = W_down.shape[1]

    y_down, stats_d = _linear_with_stats(
        x_down.reshape(B * n, Cin), W_down, b_down, blk=1024)
    mom_u = _up_moments(x_up.reshape(B * N, C), W_up, blk=2048)

    def affine(stats, gamma, beta, cnt):
        mean = stats[0] / cnt
        var = stats[1] / cnt - mean * mean
        scale = gamma / jnp.sqrt(var + 1e-5)
        return scale, beta - mean * scale

    sd, td = affine(stats_d, gamma_down, beta_down, float(B * n))
    m_u = mom_u[2] / float(B * N)
    var_u = mom_u[1] / float(B * N) - m_u * m_u
    su = gamma_up / jnp.sqrt(var_u + 1e-5)
    tu = beta_up - m_u * su
    aff = jnp.concatenate(
        [sd[None], td[None], su[None], tu[None],
         jnp.zeros((4, C), F32)], axis=0)

    pu_pad = jnp.pad(p_up, ((0, 0), (0, 0), (0, 5)))
    pd_t = jnp.pad(jnp.swapaxes(p_down, 1, 2), ((0, 0), (0, 5), (0, 0)))
    ramp = jnp.broadcast_to(
        jnp.arange(n, dtype=F32)[None, :] * jnp.float32(1.2e-38), (8, n))

    blk = 1024
    out = pl.pallas_call(
        _interp_kernel,
        grid=(B, N // blk),
        in_specs=[
            pl.BlockSpec((1, blk, 8), lambda b, i: (b, i, 0)),
            pl.BlockSpec((1, 8, n), lambda b, i: (b, 0, 0)),
            pl.BlockSpec((8, n), lambda b, i: (0, 0)),
            pl.BlockSpec((1, n, C), lambda b, i: (b, 0, 0)),
            pl.BlockSpec((1, blk, C), lambda b, i: (b, i, 0)),
            pl.BlockSpec((C, C), lambda b, i: (0, 0)),
            pl.BlockSpec((8, C), lambda b, i: (0, 0)),
        ],
        out_specs=pl.BlockSpec((1, blk, C), lambda b, i: (b, i, 0)),
        out_shape=jax.ShapeDtypeStruct((B, N, C), F32),
    )(pu_pad, pd_t, ramp, y_down.reshape(B, n, C), x_up, W_up, aff)
    return out
